# Initial kernel scaffold; baseline (speedup 1.0000x reference)
import jax
import jax.numpy as jnp
from jax.experimental import pallas as pl


def kernel(user_emb, item_emb, edge_weight, edge_src, edge_dst):
    raise NotImplementedError("write your pallas kernel here")


# speedup vs baseline: 7.6128x; 7.6128x over previous
import functools

import jax
import jax.numpy as jnp
from jax import lax
from jax.experimental import pallas as pl
from jax.experimental.pallas import tpu as pltpu
from jax.experimental.pallas import tpu_sc as plsc

USER_N = 50000
ITEM_N = 50000
NN = USER_N + ITEM_N
D = 32
HALF = NN // 2

NC = 2
NS = 16

LANE = 128
ROWS_PER_TILE = 784
CHUNK_ROWS = 4
N_CHUNKS = ROWS_PER_TILE // CHUNK_ROWS
R_PAD = ROWS_PER_TILE * NS
E_PAD = R_PAD * LANE
CHUNK_E = CHUNK_ROWS * LANE

ZCH = 448
ACC_ROWS = NS * 7 * ZCH
TRASH = HALF
READ_ROWS = 3120
READ_TAIL = HALF - NS * READ_ROWS

_mesh = plsc.VectorSubcoreMesh(core_axis_name="c", subcore_axis_name="s",
                               num_cores=NC, num_subcores=NS)


def _layer_body(ego, srcr, dstr, wr, out, acc, src_v, dst_v, adj_v, w_v,
                rows, sem):
    c = lax.axis_index("c")
    s = lax.axis_index("s")
    zero16 = jnp.zeros((16,), jnp.float32)

    @pl.loop(0, ZCH)
    def _(r):
        rows[r, pl.ds(0, 16)] = zero16
        rows[r, pl.ds(16, 16)] = zero16

    for q in range(7):
        pltpu.sync_copy(rows.at[pl.ds(0, ZCH)],
                        acc.at[pl.ds((s * 7 + q) * ZCH, ZCH)])
    plsc.subcore_barrier()

    half_base = c * HALF

    @pl.loop(0, N_CHUNKS)
    def _(q):
        row0 = s * ROWS_PER_TILE + q * CHUNK_ROWS
        pltpu.sync_copy(srcr.at[pl.ds(row0, CHUNK_ROWS)], src_v)
        pltpu.sync_copy(dstr.at[pl.ds(row0, CHUNK_ROWS)], dst_v)
        pltpu.sync_copy(wr.at[pl.ds(row0, CHUNK_ROWS)], w_v)

        cps = [pltpu.async_copy(ego.at[src_v.at[j]],
                                rows.at[pl.ds(j * LANE, LANE)], sem)
               for j in range(CHUNK_ROWS)]

        for j in range(CHUNK_ROWS):
            for i in range(LANE // 16):
                dv = dst_v[j, pl.ds(i * 16, 16)]
                lv = dv - half_base
                inr = (lv >= 0) & (lv < HALF)
                adj_v[j, pl.ds(i * 16, 16)] = jnp.where(inr, lv, TRASH)

        for cp in cps:
            cp.wait()

        @pl.loop(0, CHUNK_E // 16)
        def _(g):
            j = g >> 3
            i = (g & 7) * 16
            w16 = w_v[j, pl.ds(i, 16)]
            e0 = g * 16
            for l in range(16):
                w = w16[l]
                rows[e0 + l, pl.ds(0, 16)] = rows[e0 + l, pl.ds(0, 16)] * w
                rows[e0 + l, pl.ds(16, 16)] = rows[e0 + l, pl.ds(16, 16)] * w

        for j in range(CHUNK_ROWS):
            pltpu.sync_copy(rows.at[pl.ds(j * LANE, LANE)],
                            acc.at[adj_v.at[j]], add=True)

    plsc.subcore_barrier()
    pltpu.sync_copy(acc.at[pl.ds(s * READ_ROWS, READ_ROWS)],
                    out.at[pl.ds(c * HALF + s * READ_ROWS, READ_ROWS)])

    @pl.when(s == 0)
    def _():
        pltpu.sync_copy(acc.at[pl.ds(NS * READ_ROWS, READ_TAIL)],
                        out.at[pl.ds(c * HALF + NS * READ_ROWS, READ_TAIL)])


_layer = pl.kernel(
    _layer_body,
    out_type=jax.ShapeDtypeStruct((NN, D), jnp.float32),
    mesh=_mesh,
    scratch_types=[
        pltpu.VMEM_SHARED((ACC_ROWS, D), jnp.float32),
        pltpu.VMEM((CHUNK_ROWS, LANE), jnp.int32),
        pltpu.VMEM((CHUNK_ROWS, LANE), jnp.int32),
        pltpu.VMEM((CHUNK_ROWS, LANE), jnp.int32),
        pltpu.VMEM((CHUNK_ROWS, LANE), jnp.float32),
        pltpu.VMEM((CHUNK_E, D), jnp.float32),
        pltpu.SemaphoreType.DMA,
    ],
    compiler_params=pltpu.CompilerParams(use_tc_tiling_on_sc=False),
)

MEAN_CH = 624
MEAN_ROWS = 3120
MEAN_TAIL = NN - NC * NS * MEAN_ROWS


def _mean_body(x1, x2, x3, out, b1, b2, b3):
    c = lax.axis_index("c")
    s = lax.axis_index("s")
    wid = s * NC + c
    base = wid * MEAN_ROWS
    third = jnp.float32(1.0 / 3.0)

    def avg_rows(n_rows):
        @plsc.parallel_loop(0, n_rows * 2, 1, unroll=4)
        def _(t):
            r = t >> 1
            col = (t & 1) * 16
            v = (b1[r, pl.ds(col, 16)] + b2[r, pl.ds(col, 16)]
                 + b3[r, pl.ds(col, 16)]) * third
            b1[r, pl.ds(col, 16)] = v

    @pl.loop(0, MEAN_ROWS // MEAN_CH)
    def _(q):
        r0 = base + q * MEAN_CH
        pltpu.sync_copy(x1.at[pl.ds(r0, MEAN_CH)], b1)
        pltpu.sync_copy(x2.at[pl.ds(r0, MEAN_CH)], b2)
        pltpu.sync_copy(x3.at[pl.ds(r0, MEAN_CH)], b3)
        avg_rows(MEAN_CH)
        pltpu.sync_copy(b1, out.at[pl.ds(r0, MEAN_CH)])

    @pl.when(wid == 0)
    def _():
        t0 = NC * NS * MEAN_ROWS
        pltpu.sync_copy(x1.at[pl.ds(t0, MEAN_TAIL)], b1.at[pl.ds(0, MEAN_TAIL)])
        pltpu.sync_copy(x2.at[pl.ds(t0, MEAN_TAIL)], b2.at[pl.ds(0, MEAN_TAIL)])
        pltpu.sync_copy(x3.at[pl.ds(t0, MEAN_TAIL)], b3.at[pl.ds(0, MEAN_TAIL)])
        avg_rows(MEAN_TAIL)
        pltpu.sync_copy(b1.at[pl.ds(0, MEAN_TAIL)], out.at[pl.ds(t0, MEAN_TAIL)])


_mean = pl.kernel(
    _mean_body,
    out_type=jax.ShapeDtypeStruct((NN, D), jnp.float32),
    mesh=_mesh,
    scratch_types=[
        pltpu.VMEM((MEAN_CH, D), jnp.float32),
        pltpu.VMEM((MEAN_CH, D), jnp.float32),
        pltpu.VMEM((MEAN_CH, D), jnp.float32),
    ],
    compiler_params=pltpu.CompilerParams(use_tc_tiling_on_sc=False),
)


def kernel(user_emb, item_emb, edge_weight, edge_src, edge_dst):
    ego0 = jnp.concatenate([user_emb, item_emb], axis=0)

    pad = E_PAD - edge_src.shape[0]
    src = jnp.concatenate(
        [edge_src.astype(jnp.int32), jnp.zeros((pad,), jnp.int32)])
    dst = jnp.concatenate(
        [edge_dst.astype(jnp.int32), jnp.full((pad,), NN, jnp.int32)])
    w = jnp.concatenate([edge_weight, jnp.zeros((pad,), jnp.float32)])
    srcr = src.reshape(R_PAD, LANE)
    dstr = dst.reshape(R_PAD, LANE)
    wr = w.reshape(R_PAD, LANE)

    x1 = _layer(ego0, srcr, dstr, wr)
    x2 = _layer(x1, srcr, dstr, wr)
    x3 = _layer(x2, srcr, dstr, wr)
    final = _mean(x1, x2, x3)
    return (final[:USER_N], final[USER_N:])
